# baseline (speedup 1.0000x reference)
import jax
import jax.numpy as jnp
from jax.experimental import pallas as pl
from jax.experimental.pallas import tpu as pltpu

_BLOCK = 10000


def _copy_kernel(u_ref, i_ref, c_ref, ou_ref, oi_ref, oc_ref):
    ou_ref[...] = u_ref[...]
    oi_ref[...] = i_ref[...]
    oc_ref[...] = c_ref[...]


def kernel(emb_user, emb_item, emb_category):
    n, d = emb_user.shape
    grid = (n // _BLOCK,)
    big_spec = pl.BlockSpec((_BLOCK, d), lambda i: (i, 0))
    cat_spec = pl.BlockSpec(emb_category.shape, lambda i: (0, 0))
    outs = pl.pallas_call(
        _copy_kernel,
        grid=grid,
        out_shape=tuple(
            jax.ShapeDtypeStruct(x.shape, x.dtype)
            for x in (emb_user, emb_item, emb_category)
        ),
        in_specs=[big_spec, big_spec, cat_spec],
        out_specs=[big_spec, big_spec, cat_spec],
        compiler_params=pltpu.CompilerParams(
            dimension_semantics=("parallel",)),
    )(emb_user, emb_item, emb_category)
    return outs
